# Initial kernel scaffold; baseline (speedup 1.0000x reference)
import jax
import jax.numpy as jnp
from jax.experimental import pallas as pl


def kernel(word_ids, table):
    raise NotImplementedError("write your pallas kernel here")


# speedup vs baseline: 1.4757x; 1.4757x over previous
import functools

import jax
import jax.numpy as jnp
from jax import lax
from jax.experimental import pallas as pl
from jax.experimental.pallas import tpu as pltpu
from jax.experimental.pallas import tpu_sc as plsc

EMB = 32
N = 4096 * 200
NW = 32
PER_W = N // NW
CHUNK = 1600
NCHUNK = PER_W // CHUNK


def _make_gather():
    mesh = plsc.VectorSubcoreMesh(core_axis_name="c", subcore_axis_name="s")

    @functools.partial(
        pl.kernel,
        mesh=mesh,
        out_type=jax.ShapeDtypeStruct((N, EMB), jnp.float32),
        scratch_types=[
            pltpu.VMEM((CHUNK,), jnp.int32),
            pltpu.VMEM((CHUNK, EMB), jnp.float32),
            pltpu.SemaphoreType.DMA,
        ],
        compiler_params=pltpu.CompilerParams(use_tc_tiling_on_sc=False),
    )
    def gather_kernel(ids_hbm, table_hbm, out_hbm, idx_v, rows_v, sem):
        wid = lax.axis_index("s") * 2 + lax.axis_index("c")
        base = wid * PER_W

        def body(i, carry):
            off = base + i * CHUNK
            pltpu.sync_copy(ids_hbm.at[pl.ds(off, CHUNK)], idx_v)
            pltpu.async_copy(table_hbm.at[idx_v], rows_v, sem).wait()
            pltpu.sync_copy(rows_v, out_hbm.at[pl.ds(off, CHUNK)])
            return carry

        lax.fori_loop(0, NCHUNK, body, 0)

    return gather_kernel


_gather = _make_gather()


def kernel(word_ids, table):
    flat = word_ids.reshape(-1)
    out = _gather(flat, table)
    return out.reshape(word_ids.shape + (EMB,))


# speedup vs baseline: 1.5011x; 1.0172x over previous
import functools

import jax
import jax.numpy as jnp
from jax import lax
from jax.experimental import pallas as pl
from jax.experimental.pallas import tpu as pltpu
from jax.experimental.pallas import tpu_sc as plsc

EMB = 32
N = 4096 * 200
NW = 32
PER_W = N // NW
NBUF = 4
CHUNK = 800
NCHUNK = PER_W // CHUNK
NOUTER = NCHUNK // NBUF


def _make_gather():
    mesh = plsc.VectorSubcoreMesh(core_axis_name="c", subcore_axis_name="s")

    scratch = (
        [pltpu.VMEM((CHUNK,), jnp.int32) for _ in range(NBUF)]
        + [pltpu.VMEM((CHUNK, EMB), jnp.float32) for _ in range(NBUF)]
        + [pltpu.SemaphoreType.DMA for _ in range(3 * NBUF)]
    )

    @functools.partial(
        pl.kernel,
        mesh=mesh,
        out_type=jax.ShapeDtypeStruct((N, EMB), jnp.float32),
        scratch_types=scratch,
        compiler_params=pltpu.CompilerParams(use_tc_tiling_on_sc=False),
    )
    def gather_kernel(ids_hbm, table_hbm, out_hbm, *scratch_refs):
        idx_v = scratch_refs[:NBUF]
        rows_v = scratch_refs[NBUF:2 * NBUF]
        isem = scratch_refs[2 * NBUF:3 * NBUF]
        gsem = scratch_refs[3 * NBUF:4 * NBUF]
        wsem = scratch_refs[4 * NBUF:5 * NBUF]

        wid = lax.axis_index("s") * 2 + lax.axis_index("c")
        base = wid * PER_W

        def idx_start(b, i):
            pltpu.async_copy(
                ids_hbm.at[pl.ds(base + i * CHUNK, CHUNK)], idx_v[b], isem[b])

        def idx_wait(b, i):
            pltpu.make_async_copy(
                ids_hbm.at[pl.ds(base + i * CHUNK, CHUNK)], idx_v[b],
                isem[b]).wait()

        def gather_start(b):
            pltpu.async_copy(table_hbm.at[idx_v[b]], rows_v[b], gsem[b])

        def gather_wait(b):
            pltpu.make_async_copy(
                table_hbm.at[idx_v[b]], rows_v[b], gsem[b]).wait()

        def write_start(b, i):
            pltpu.async_copy(
                rows_v[b], out_hbm.at[pl.ds(base + i * CHUNK, CHUNK)], wsem[b])

        def write_wait(b, i):
            pltpu.make_async_copy(
                rows_v[b], out_hbm.at[pl.ds(base + i * CHUNK, CHUNK)],
                wsem[b]).wait()

        for b in range(NBUF):
            idx_start(b, b)
        for b in range(NBUF):
            idx_wait(b, b)
            gather_start(b)

        def outer_body(g, carry):
            for b in range(NBUF):
                i = g * NBUF + b
                j = i + NBUF
                gather_wait(b)
                idx_start(b, j)
                write_start(b, i)
                write_wait(b, i)
                idx_wait(b, j)
                gather_start(b)
            return carry

        lax.fori_loop(0, NOUTER - 1, outer_body, 0)

        last = (NOUTER - 1) * NBUF
        for b in range(NBUF):
            gather_wait(b)
            write_start(b, last + b)
        for b in range(NBUF):
            write_wait(b, last + b)

    return gather_kernel


_gather = _make_gather()


def kernel(word_ids, table):
    flat = word_ids.reshape(-1)
    out = _gather(flat, table)
    return out.reshape(word_ids.shape + (EMB,))
